# Initial kernel scaffold; baseline (speedup 1.0000x reference)
import jax
import jax.numpy as jnp
from jax.experimental import pallas as pl


def kernel(x, edge_index, W1, as1, ad1, b1, W2, as2, ad2, b2, W3, as3, ad3, b3):
    raise NotImplementedError("write your pallas kernel here")


# speedup vs baseline: 19.9580x; 19.9580x over previous
import functools

import jax
import jax.numpy as jnp
from jax import lax
from jax.experimental import pallas as pl
from jax.experimental.pallas import tpu as pltpu
from jax.experimental.pallas import tpu_sc as plsc

N = 10000
E = 320000
D = 128

N_PAD = 10240
NC = 2
NS = 16
NW = NC * NS
CHUNK = 128
ROWS_PER_TILE = N_PAD // NS
E_PAD = ((E + NW * CHUNK - 1) // (NW * CHUNK)) * (NW * CHUNK)
CPW = E_PAD // (NW * CHUNK)
RB = 1024
GRID = N_PAD // RB


def _dense_first_body(x_ref, w_ref, ats_ref, atd_ref, h_ref, as_ref, ad_ref):
    h = jnp.dot(x_ref[...], w_ref[...], preferred_element_type=jnp.float32)
    h_ref[...] = h
    as_ref[...] = jnp.sum(h * ats_ref[...], axis=1, keepdims=True)
    ad_ref[...] = jnp.sum(h * atd_ref[...], axis=1, keepdims=True)


def _dense_first(x, W, ats, atd):
    return pl.pallas_call(
        _dense_first_body,
        grid=(GRID,),
        in_specs=[
            pl.BlockSpec((RB, D), lambda i: (i, 0)),
            pl.BlockSpec((D, D), lambda i: (0, 0)),
            pl.BlockSpec((1, D), lambda i: (0, 0)),
            pl.BlockSpec((1, D), lambda i: (0, 0)),
        ],
        out_specs=[
            pl.BlockSpec((RB, D), lambda i: (i, 0)),
            pl.BlockSpec((RB, 1), lambda i: (i, 0)),
            pl.BlockSpec((RB, 1), lambda i: (i, 0)),
        ],
        out_shape=[
            jax.ShapeDtypeStruct((N_PAD, D), jnp.float32),
            jax.ShapeDtypeStruct((N_PAD, 1), jnp.float32),
            jax.ShapeDtypeStruct((N_PAD, 1), jnp.float32),
        ],
    )(x, W, ats, atd)


def _finish_node(acc_ref, den_ref, hp_ref, asp_ref, adp_ref, b_ref):
    e = asp_ref[...] + adp_ref[...]
    ws = jnp.exp(jnp.maximum(e, 0.2 * e))
    num = acc_ref[0] + acc_ref[1] + ws * hp_ref[...]
    den = den_ref[0] + den_ref[1] + ws
    return num / den + b_ref[...]


def _dense_mid_body(acc_ref, den_ref, hp_ref, asp_ref, adp_ref, b_ref,
                    w_ref, ats_ref, atd_ref, h_ref, as_ref, ad_ref):
    x = jnp.maximum(_finish_node(acc_ref, den_ref, hp_ref, asp_ref, adp_ref, b_ref), 0.0)
    h = jnp.dot(x, w_ref[...], preferred_element_type=jnp.float32)
    h_ref[...] = h
    as_ref[...] = jnp.sum(h * ats_ref[...], axis=1, keepdims=True)
    ad_ref[...] = jnp.sum(h * atd_ref[...], axis=1, keepdims=True)


def _dense_mid(acc, den, hp, asp, adp, b, W, ats, atd):
    return pl.pallas_call(
        _dense_mid_body,
        grid=(GRID,),
        in_specs=[
            pl.BlockSpec((NC, RB, D), lambda i: (0, i, 0)),
            pl.BlockSpec((NC, RB, 1), lambda i: (0, i, 0)),
            pl.BlockSpec((RB, D), lambda i: (i, 0)),
            pl.BlockSpec((RB, 1), lambda i: (i, 0)),
            pl.BlockSpec((RB, 1), lambda i: (i, 0)),
            pl.BlockSpec((1, D), lambda i: (0, 0)),
            pl.BlockSpec((D, D), lambda i: (0, 0)),
            pl.BlockSpec((1, D), lambda i: (0, 0)),
            pl.BlockSpec((1, D), lambda i: (0, 0)),
        ],
        out_specs=[
            pl.BlockSpec((RB, D), lambda i: (i, 0)),
            pl.BlockSpec((RB, 1), lambda i: (i, 0)),
            pl.BlockSpec((RB, 1), lambda i: (i, 0)),
        ],
        out_shape=[
            jax.ShapeDtypeStruct((N_PAD, D), jnp.float32),
            jax.ShapeDtypeStruct((N_PAD, 1), jnp.float32),
            jax.ShapeDtypeStruct((N_PAD, 1), jnp.float32),
        ],
    )(acc, den, hp, asp, adp, b, W, ats, atd)


def _dense_last_body(acc_ref, den_ref, hp_ref, asp_ref, adp_ref, b_ref, out_ref):
    out_ref[...] = _finish_node(acc_ref, den_ref, hp_ref, asp_ref, adp_ref, b_ref)


def _dense_last(acc, den, hp, asp, adp, b):
    return pl.pallas_call(
        _dense_last_body,
        grid=(GRID,),
        in_specs=[
            pl.BlockSpec((NC, RB, D), lambda i: (0, i, 0)),
            pl.BlockSpec((NC, RB, 1), lambda i: (0, i, 0)),
            pl.BlockSpec((RB, D), lambda i: (i, 0)),
            pl.BlockSpec((RB, 1), lambda i: (i, 0)),
            pl.BlockSpec((RB, 1), lambda i: (i, 0)),
            pl.BlockSpec((1, D), lambda i: (0, 0)),
        ],
        out_specs=[pl.BlockSpec((RB, D), lambda i: (i, 0))],
        out_shape=[jax.ShapeDtypeStruct((N_PAD, D), jnp.float32)],
    )(acc, den, hp, asp, adp, b)


_SC_MESH = plsc.VectorSubcoreMesh(core_axis_name="c", subcore_axis_name="s")


@functools.partial(
    pl.kernel,
    out_type=[
        jax.ShapeDtypeStruct((NC, N_PAD, D), jnp.float32),
        jax.ShapeDtypeStruct((NC, N_PAD), jnp.float32),
    ],
    mesh=_SC_MESH,
    compiler_params=pltpu.CompilerParams(needs_layout_passes=False),
    scratch_types=[
        pltpu.VMEM((N_PAD,), jnp.float32),
        pltpu.VMEM((N_PAD,), jnp.float32),
        pltpu.VMEM((CHUNK,), jnp.int32),
        pltpu.VMEM((CHUNK,), jnp.int32),
        pltpu.VMEM((CHUNK,), jnp.float32),
        pltpu.VMEM((CHUNK, D), jnp.float32),
        pltpu.VMEM_SHARED((N_PAD, D), jnp.float32),
        pltpu.VMEM_SHARED((N_PAD,), jnp.float32),
        pltpu.SemaphoreType.DMA,
    ],
)
def _sc_edge_agg(h_hbm, as_hbm, ad_hbm, src_hbm, dst_hbm,
                 acc_out, den_out,
                 as_v, ad_v, srcb, dstb, wb, rowsb, accS, denS, sem):
    cid = lax.axis_index("c")
    sid = lax.axis_index("s")
    wid = cid * NS + sid
    zero16 = jnp.zeros((16,), jnp.float32)

    @pl.loop(0, CHUNK)
    def _(r):
        for c in range(D // 16):
            rowsb[r, pl.ds(16 * c, 16)] = zero16

    for c in range(CHUNK // 16):
        wb[pl.ds(16 * c, 16)] = zero16

    row0 = sid * ROWS_PER_TILE
    for k in range(ROWS_PER_TILE // CHUNK):
        pltpu.sync_copy(rowsb, accS.at[pl.ds(row0 + k * CHUNK, CHUNK)])
        pltpu.sync_copy(wb, denS.at[pl.ds(row0 + k * CHUNK, CHUNK)])

    pltpu.sync_copy(as_hbm, as_v)
    pltpu.sync_copy(ad_hbm, ad_v)

    plsc.subcore_barrier()

    @pl.loop(0, CPW)
    def _(t):
        base = (wid * CPW + t) * CHUNK
        pltpu.sync_copy(src_hbm.at[pl.ds(base, CHUNK)], srcb)
        pltpu.sync_copy(dst_hbm.at[pl.ds(base, CHUNK)], dstb)

        gather = pltpu.async_copy(h_hbm.at[srcb], rowsb, sem)

        for k in range(CHUNK // 16):
            sl = pl.ds(16 * k, 16)
            i_s = srcb[sl]
            i_d = dstb[sl]
            e = plsc.load_gather(as_v, [i_s]) + plsc.load_gather(ad_v, [i_d])
            wb[sl] = jnp.exp(jnp.maximum(e, 0.2 * e))

        gather.wait()

        @pl.loop(0, CHUNK)
        def _(j):
            wj = plsc.load_gather(wb, [jnp.full((16,), j, jnp.int32)])
            for c in range(D // 16):
                sl = pl.ds(16 * c, 16)
                rowsb[j, sl] = rowsb[j, sl] * wj

        pltpu.sync_copy(rowsb, accS.at[dstb], add=True)
        pltpu.sync_copy(wb, denS.at[dstb], add=True)

    plsc.subcore_barrier()

    pltpu.sync_copy(accS.at[pl.ds(row0, ROWS_PER_TILE)],
                    acc_out.at[cid, pl.ds(row0, ROWS_PER_TILE)])
    pltpu.sync_copy(denS.at[pl.ds(row0, ROWS_PER_TILE)],
                    den_out.at[cid, pl.ds(row0, ROWS_PER_TILE)])


def kernel(x, edge_index, W1, as1, ad1, b1, W2, as2, ad2, b2, W3, as3, ad3, b3):
    ei = edge_index.astype(jnp.int32)
    pad = jnp.full((E_PAD - E,), N, jnp.int32)
    src = jnp.concatenate([ei[0], pad])
    dst = jnp.concatenate([ei[1], pad])

    xp = jnp.pad(x, ((0, N_PAD - N), (0, 0)))

    def edge_phase(h, a_s, a_d):
        acc, den = _sc_edge_agg(h, a_s.reshape(N_PAD), a_d.reshape(N_PAD),
                                src, dst)
        return acc, den.reshape(NC, N_PAD, 1)

    def r2(v):
        return v.reshape(1, D)

    h1, s1, d1 = _dense_first(xp, W1, r2(as1), r2(ad1))
    acc1, den1 = edge_phase(h1, s1, d1)
    h2, s2, d2 = _dense_mid(acc1, den1, h1, s1, d1, r2(b1), W2, r2(as2), r2(ad2))
    acc2, den2 = edge_phase(h2, s2, d2)
    h3, s3, d3 = _dense_mid(acc2, den2, h2, s2, d2, r2(b2), W3, r2(as3), r2(ad3))
    acc3, den3 = edge_phase(h3, s3, d3)
    (out,) = _dense_last(acc3, den3, h3, s3, d3, r2(b3))
    return out[:N]


# speedup vs baseline: 22.0589x; 1.1053x over previous
import functools

import jax
import jax.numpy as jnp
from jax import lax
from jax.experimental import pallas as pl
from jax.experimental.pallas import tpu as pltpu
from jax.experimental.pallas import tpu_sc as plsc

N = 10000
E = 320000
D = 128

N_PAD = 10240
NC = 2
NS = 16
NW = NC * NS
CHUNK = 64
NB = 16
NBLK = 10
CPW = NB * NBLK
ROWS_PER_TILE = N_PAD // NS
E_PAD = NW * CPW * CHUNK
NCHUNKS = E_PAD // CHUNK
RB = 1024
GRID = N_PAD // RB


def _dense_first_body(x_ref, w_ref, ats_ref, atd_ref, h_ref, as_ref, ad_ref):
    h = jnp.dot(x_ref[...], w_ref[...], preferred_element_type=jnp.float32)
    h_ref[...] = h
    as_ref[...] = jnp.sum(h * ats_ref[...], axis=1, keepdims=True)
    ad_ref[...] = jnp.sum(h * atd_ref[...], axis=1, keepdims=True)


def _dense_first(x, W, ats, atd):
    return pl.pallas_call(
        _dense_first_body,
        grid=(GRID,),
        in_specs=[
            pl.BlockSpec((RB, D), lambda i: (i, 0)),
            pl.BlockSpec((D, D), lambda i: (0, 0)),
            pl.BlockSpec((1, D), lambda i: (0, 0)),
            pl.BlockSpec((1, D), lambda i: (0, 0)),
        ],
        out_specs=[
            pl.BlockSpec((RB, D), lambda i: (i, 0)),
            pl.BlockSpec((RB, 1), lambda i: (i, 0)),
            pl.BlockSpec((RB, 1), lambda i: (i, 0)),
        ],
        out_shape=[
            jax.ShapeDtypeStruct((N_PAD, D), jnp.float32),
            jax.ShapeDtypeStruct((N_PAD, 1), jnp.float32),
            jax.ShapeDtypeStruct((N_PAD, 1), jnp.float32),
        ],
    )(x, W, ats, atd)


def _finish_node(acc_ref, den_ref, hp_ref, asp_ref, adp_ref, b_ref):
    e = asp_ref[...] + adp_ref[...]
    ws = jnp.exp(jnp.maximum(e, 0.2 * e))
    num = acc_ref[0] + acc_ref[1] + ws * hp_ref[...]
    den = den_ref[0] + den_ref[1] + ws
    return num / den + b_ref[...]


def _dense_mid_body(acc_ref, den_ref, hp_ref, asp_ref, adp_ref, b_ref,
                    w_ref, ats_ref, atd_ref, h_ref, as_ref, ad_ref):
    x = jnp.maximum(_finish_node(acc_ref, den_ref, hp_ref, asp_ref, adp_ref, b_ref), 0.0)
    h = jnp.dot(x, w_ref[...], preferred_element_type=jnp.float32)
    h_ref[...] = h
    as_ref[...] = jnp.sum(h * ats_ref[...], axis=1, keepdims=True)
    ad_ref[...] = jnp.sum(h * atd_ref[...], axis=1, keepdims=True)


def _dense_mid(acc, den, hp, asp, adp, b, W, ats, atd):
    return pl.pallas_call(
        _dense_mid_body,
        grid=(GRID,),
        in_specs=[
            pl.BlockSpec((NC, RB, D), lambda i: (0, i, 0)),
            pl.BlockSpec((NC, RB, 1), lambda i: (0, i, 0)),
            pl.BlockSpec((RB, D), lambda i: (i, 0)),
            pl.BlockSpec((RB, 1), lambda i: (i, 0)),
            pl.BlockSpec((RB, 1), lambda i: (i, 0)),
            pl.BlockSpec((1, D), lambda i: (0, 0)),
            pl.BlockSpec((D, D), lambda i: (0, 0)),
            pl.BlockSpec((1, D), lambda i: (0, 0)),
            pl.BlockSpec((1, D), lambda i: (0, 0)),
        ],
        out_specs=[
            pl.BlockSpec((RB, D), lambda i: (i, 0)),
            pl.BlockSpec((RB, 1), lambda i: (i, 0)),
            pl.BlockSpec((RB, 1), lambda i: (i, 0)),
        ],
        out_shape=[
            jax.ShapeDtypeStruct((N_PAD, D), jnp.float32),
            jax.ShapeDtypeStruct((N_PAD, 1), jnp.float32),
            jax.ShapeDtypeStruct((N_PAD, 1), jnp.float32),
        ],
    )(acc, den, hp, asp, adp, b, W, ats, atd)


def _dense_last_body(acc_ref, den_ref, hp_ref, asp_ref, adp_ref, b_ref, out_ref):
    out_ref[...] = _finish_node(acc_ref, den_ref, hp_ref, asp_ref, adp_ref, b_ref)


def _dense_last(acc, den, hp, asp, adp, b):
    return pl.pallas_call(
        _dense_last_body,
        grid=(GRID,),
        in_specs=[
            pl.BlockSpec((NC, RB, D), lambda i: (0, i, 0)),
            pl.BlockSpec((NC, RB, 1), lambda i: (0, i, 0)),
            pl.BlockSpec((RB, D), lambda i: (i, 0)),
            pl.BlockSpec((RB, 1), lambda i: (i, 0)),
            pl.BlockSpec((RB, 1), lambda i: (i, 0)),
            pl.BlockSpec((1, D), lambda i: (0, 0)),
        ],
        out_specs=[pl.BlockSpec((RB, D), lambda i: (i, 0))],
        out_shape=[jax.ShapeDtypeStruct((N_PAD, D), jnp.float32)],
    )(acc, den, hp, asp, adp, b)


_SC_MESH = plsc.VectorSubcoreMesh(core_axis_name="c", subcore_axis_name="s")


@functools.partial(
    pl.kernel,
    out_type=[
        jax.ShapeDtypeStruct((NC, N_PAD, D), jnp.float32),
        jax.ShapeDtypeStruct((NC, N_PAD), jnp.float32),
    ],
    mesh=_SC_MESH,
    compiler_params=pltpu.CompilerParams(needs_layout_passes=False),
    scratch_types=[
        pltpu.VMEM((N_PAD,), jnp.float32),
        pltpu.VMEM((N_PAD,), jnp.float32),
        pltpu.VMEM((NB, CHUNK), jnp.int32),
        pltpu.VMEM((NB, CHUNK), jnp.int32),
        pltpu.VMEM((NB, CHUNK), jnp.float32),
        pltpu.VMEM((CHUNK, D), jnp.float32),
        pltpu.VMEM((CHUNK, D), jnp.float32),
        pltpu.VMEM_SHARED((N_PAD, D), jnp.float32),
        pltpu.VMEM_SHARED((N_PAD,), jnp.float32),
        pltpu.SemaphoreType.DMA,
        pltpu.SemaphoreType.DMA,
    ],
)
def _sc_edge_agg(h_hbm, as_hbm, ad_hbm, src_hbm, dst_hbm,
                 acc_out, den_out,
                 as_v, ad_v, srcb, dstb, wb, rows0, rows1, accS, denS,
                 gs0, gs1):
    cid = lax.axis_index("c")
    sid = lax.axis_index("s")
    wid = cid * NS + sid
    zero16 = jnp.zeros((16,), jnp.float32)

    pltpu.sync_copy(as_hbm, as_v)
    pltpu.sync_copy(ad_hbm, ad_v)

    @pl.loop(0, CHUNK)
    def _(r):
        for c in range(D // 16):
            rows0[r, pl.ds(16 * c, 16)] = zero16

    row0 = sid * ROWS_PER_TILE
    for k in range(ROWS_PER_TILE // CHUNK):
        pltpu.sync_copy(rows0, accS.at[pl.ds(row0 + k * CHUNK, CHUNK)])
    for k in range(ROWS_PER_TILE // D):
        pltpu.sync_copy(rows0.at[0], denS.at[pl.ds(row0 + k * D, D)])

    plsc.subcore_barrier()

    def _scale(rows, wrow):
        @pl.loop(0, CHUNK, unroll=2)
        def _(j):
            wj = plsc.load_gather(wrow, [jnp.full((16,), j, jnp.int32)])
            for c in range(D // 16):
                sl = pl.ds(16 * c, 16)
                rows[j, sl] = rows[j, sl] * wj

    def _scatter(rows, q2):
        pltpu.sync_copy(rows, accS.at[dstb.at[q2]], add=True)
        pltpu.sync_copy(wb.at[q2], denS.at[dstb.at[q2]], add=True)

    @pl.loop(0, NBLK)
    def _(b):
        blk = wid * NBLK + b
        pltpu.sync_copy(src_hbm.at[pl.ds(blk * NB, NB)], srcb)
        pltpu.sync_copy(dst_hbm.at[pl.ds(blk * NB, NB)], dstb)

        @pl.loop(0, NB)
        def _(q2):
            for k in range(CHUNK // 16):
                sl = pl.ds(16 * k, 16)
                e = (plsc.load_gather(as_v, [srcb[q2, sl]])
                     + plsc.load_gather(ad_v, [dstb[q2, sl]]))
                wb[q2, sl] = jnp.exp(jnp.maximum(e, 0.2 * e))

        pltpu.async_copy(h_hbm.at[srcb.at[0]], rows0, gs0)

        @pl.loop(0, NB, step=2)
        def _(q2):
            pltpu.async_copy(h_hbm.at[srcb.at[q2 + 1]], rows1, gs1)
            pltpu.make_async_copy(h_hbm.at[srcb.at[q2]], rows0, gs0).wait()
            _scale(rows0, wb.at[q2])
            _scatter(rows0, q2)

            @pl.when(q2 + 2 < NB)
            def _():
                pltpu.async_copy(h_hbm.at[srcb.at[q2 + 2]], rows0, gs0)

            pltpu.make_async_copy(h_hbm.at[srcb.at[q2 + 1]], rows1, gs1).wait()
            _scale(rows1, wb.at[q2 + 1])
            _scatter(rows1, q2 + 1)

    plsc.subcore_barrier()

    pltpu.sync_copy(accS.at[pl.ds(row0, ROWS_PER_TILE)],
                    acc_out.at[cid, pl.ds(row0, ROWS_PER_TILE)])
    pltpu.sync_copy(denS.at[pl.ds(row0, ROWS_PER_TILE)],
                    den_out.at[cid, pl.ds(row0, ROWS_PER_TILE)])


def kernel(x, edge_index, W1, as1, ad1, b1, W2, as2, ad2, b2, W3, as3, ad3, b3):
    ei = edge_index.astype(jnp.int32)
    pad = jnp.full((E_PAD - E,), N, jnp.int32)
    src = jnp.concatenate([ei[0], pad]).reshape(NCHUNKS, CHUNK)
    dst = jnp.concatenate([ei[1], pad]).reshape(NCHUNKS, CHUNK)

    xp = jnp.pad(x, ((0, N_PAD - N), (0, 0)))

    def edge_phase(h, a_s, a_d):
        acc, den = _sc_edge_agg(h, a_s.reshape(N_PAD), a_d.reshape(N_PAD),
                                src, dst)
        return acc, den.reshape(NC, N_PAD, 1)

    def r2(v):
        return v.reshape(1, D)

    h1, s1, d1 = _dense_first(xp, W1, r2(as1), r2(ad1))
    acc1, den1 = edge_phase(h1, s1, d1)
    h2, s2, d2 = _dense_mid(acc1, den1, h1, s1, d1, r2(b1), W2, r2(as2), r2(ad2))
    acc2, den2 = edge_phase(h2, s2, d2)
    h3, s3, d3 = _dense_mid(acc2, den2, h2, s2, d2, r2(b2), W3, r2(as3), r2(ad3))
    acc3, den3 = edge_phase(h3, s3, d3)
    (out,) = _dense_last(acc3, den3, h3, s3, d3, r2(b3))
    return out[:N]


# speedup vs baseline: 24.1499x; 1.0948x over previous
import functools

import jax
import jax.numpy as jnp
from jax import lax
from jax.experimental import pallas as pl
from jax.experimental.pallas import tpu as pltpu
from jax.experimental.pallas import tpu_sc as plsc

N = 10000
E = 320000
D = 128

N_PAD = 10240
NC = 2
NS = 16
NW = NC * NS
CHUNK = 64
NB = 16
NBLK0 = 13
NBLK1 = 7
ROWS_PER_TILE = N_PAD // NS
E_PAD = NS * (NBLK0 + NBLK1) * NB * CHUNK
NCHUNKS = E_PAD // CHUNK
RB = 1024
GRID = N_PAD // RB


def _dense_first_body(x_ref, w_ref, ats_ref, atd_ref, h_ref, as_ref, ad_ref):
    h = jnp.dot(x_ref[...], w_ref[...], preferred_element_type=jnp.float32)
    h_ref[...] = h
    as_ref[...] = jnp.sum(h * ats_ref[...], axis=1, keepdims=True)
    ad_ref[...] = jnp.sum(h * atd_ref[...], axis=1, keepdims=True)


def _dense_first(x, W, ats, atd):
    return pl.pallas_call(
        _dense_first_body,
        grid=(GRID,),
        in_specs=[
            pl.BlockSpec((RB, D), lambda i: (i, 0)),
            pl.BlockSpec((D, D), lambda i: (0, 0)),
            pl.BlockSpec((1, D), lambda i: (0, 0)),
            pl.BlockSpec((1, D), lambda i: (0, 0)),
        ],
        out_specs=[
            pl.BlockSpec((RB, D), lambda i: (i, 0)),
            pl.BlockSpec((RB, 1), lambda i: (i, 0)),
            pl.BlockSpec((RB, 1), lambda i: (i, 0)),
        ],
        out_shape=[
            jax.ShapeDtypeStruct((N_PAD, D), jnp.float32),
            jax.ShapeDtypeStruct((N_PAD, 1), jnp.float32),
            jax.ShapeDtypeStruct((N_PAD, 1), jnp.float32),
        ],
    )(x, W, ats, atd)


def _finish_node(acc_ref, den_ref, hp_ref, asp_ref, adp_ref, b_ref):
    e = asp_ref[...] + adp_ref[...]
    ws = jnp.exp(jnp.maximum(e, 0.2 * e))
    num = acc_ref[0] + acc_ref[1] + ws * hp_ref[...]
    den = den_ref[0] + den_ref[1] + ws
    return num / den + b_ref[...]


def _dense_mid_body(acc_ref, den_ref, hp_ref, asp_ref, adp_ref, b_ref,
                    w_ref, ats_ref, atd_ref, h_ref, as_ref, ad_ref):
    x = jnp.maximum(_finish_node(acc_ref, den_ref, hp_ref, asp_ref, adp_ref, b_ref), 0.0)
    h = jnp.dot(x, w_ref[...], preferred_element_type=jnp.float32)
    h_ref[...] = h
    as_ref[...] = jnp.sum(h * ats_ref[...], axis=1, keepdims=True)
    ad_ref[...] = jnp.sum(h * atd_ref[...], axis=1, keepdims=True)


def _dense_mid(acc, den, hp, asp, adp, b, W, ats, atd):
    return pl.pallas_call(
        _dense_mid_body,
        grid=(GRID,),
        in_specs=[
            pl.BlockSpec((NC, RB, D), lambda i: (0, i, 0)),
            pl.BlockSpec((NC, RB, 1), lambda i: (0, i, 0)),
            pl.BlockSpec((RB, D), lambda i: (i, 0)),
            pl.BlockSpec((RB, 1), lambda i: (i, 0)),
            pl.BlockSpec((RB, 1), lambda i: (i, 0)),
            pl.BlockSpec((1, D), lambda i: (0, 0)),
            pl.BlockSpec((D, D), lambda i: (0, 0)),
            pl.BlockSpec((1, D), lambda i: (0, 0)),
            pl.BlockSpec((1, D), lambda i: (0, 0)),
        ],
        out_specs=[
            pl.BlockSpec((RB, D), lambda i: (i, 0)),
            pl.BlockSpec((RB, 1), lambda i: (i, 0)),
            pl.BlockSpec((RB, 1), lambda i: (i, 0)),
        ],
        out_shape=[
            jax.ShapeDtypeStruct((N_PAD, D), jnp.float32),
            jax.ShapeDtypeStruct((N_PAD, 1), jnp.float32),
            jax.ShapeDtypeStruct((N_PAD, 1), jnp.float32),
        ],
    )(acc, den, hp, asp, adp, b, W, ats, atd)


def _dense_last_body(acc_ref, den_ref, hp_ref, asp_ref, adp_ref, b_ref, out_ref):
    out_ref[...] = _finish_node(acc_ref, den_ref, hp_ref, asp_ref, adp_ref, b_ref)


def _dense_last(acc, den, hp, asp, adp, b):
    return pl.pallas_call(
        _dense_last_body,
        grid=(GRID,),
        in_specs=[
            pl.BlockSpec((NC, RB, D), lambda i: (0, i, 0)),
            pl.BlockSpec((NC, RB, 1), lambda i: (0, i, 0)),
            pl.BlockSpec((RB, D), lambda i: (i, 0)),
            pl.BlockSpec((RB, 1), lambda i: (i, 0)),
            pl.BlockSpec((RB, 1), lambda i: (i, 0)),
            pl.BlockSpec((1, D), lambda i: (0, 0)),
        ],
        out_specs=[pl.BlockSpec((RB, D), lambda i: (i, 0))],
        out_shape=[jax.ShapeDtypeStruct((N_PAD, D), jnp.float32)],
    )(acc, den, hp, asp, adp, b)


_SC_MESH = plsc.VectorSubcoreMesh(core_axis_name="c", subcore_axis_name="s")


@functools.partial(
    pl.kernel,
    out_type=[
        jax.ShapeDtypeStruct((NC, N_PAD, D), jnp.float32),
        jax.ShapeDtypeStruct((NC, N_PAD), jnp.float32),
    ],
    mesh=_SC_MESH,
    compiler_params=pltpu.CompilerParams(needs_layout_passes=False),
    scratch_types=[
        pltpu.VMEM((N_PAD,), jnp.float32),
        pltpu.VMEM((N_PAD,), jnp.float32),
        pltpu.VMEM((NB, CHUNK), jnp.int32),
        pltpu.VMEM((NB, CHUNK), jnp.int32),
        pltpu.VMEM((NB, CHUNK), jnp.float32),
        pltpu.VMEM((CHUNK, D), jnp.float32),
        pltpu.VMEM((CHUNK, D), jnp.float32),
        pltpu.VMEM_SHARED((N_PAD, D), jnp.float32),
        pltpu.VMEM_SHARED((N_PAD,), jnp.float32),
        pltpu.SemaphoreType.DMA,
        pltpu.SemaphoreType.DMA,
    ],
)
def _sc_edge_agg(h_hbm, as_hbm, ad_hbm, src_hbm, dst_hbm,
                 acc_out, den_out,
                 as_v, ad_v, srcb, dstb, wb, rows0, rows1, accS, denS,
                 gs0, gs1):
    cid = lax.axis_index("c")
    sid = lax.axis_index("s")
    wid = cid * NS + sid
    zero16 = jnp.zeros((16,), jnp.float32)

    pltpu.sync_copy(as_hbm, as_v)
    pltpu.sync_copy(ad_hbm, ad_v)

    @pl.loop(0, CHUNK)
    def _(r):
        for c in range(D // 16):
            rows0[r, pl.ds(16 * c, 16)] = zero16

    row0 = sid * ROWS_PER_TILE
    for k in range(ROWS_PER_TILE // CHUNK):
        pltpu.sync_copy(rows0, accS.at[pl.ds(row0 + k * CHUNK, CHUNK)])
    for k in range(ROWS_PER_TILE // D):
        pltpu.sync_copy(rows0.at[0], denS.at[pl.ds(row0 + k * D, D)])

    plsc.subcore_barrier()

    def _scale(rows, wrow):
        @pl.loop(0, CHUNK, unroll=2)
        def _(j):
            wj = plsc.load_gather(wrow, [jnp.full((16,), j, jnp.int32)])
            for c in range(D // 16):
                sl = pl.ds(16 * c, 16)
                rows[j, sl] = rows[j, sl] * wj

    def _scatter(rows, q2):
        pltpu.sync_copy(rows, accS.at[dstb.at[q2]], add=True)
        pltpu.sync_copy(wb.at[q2], denS.at[dstb.at[q2]], add=True)

    nblk = lax.select(cid == 0, NBLK0, NBLK1)
    blk0 = lax.select(cid == 0, sid * NBLK0, NS * NBLK0 + sid * NBLK1)

    @pl.loop(0, nblk)
    def _(b):
        blk = blk0 + b
        pltpu.sync_copy(src_hbm.at[pl.ds(blk * NB, NB)], srcb)
        pltpu.sync_copy(dst_hbm.at[pl.ds(blk * NB, NB)], dstb)

        @pl.loop(0, NB)
        def _(q2):
            for k in range(CHUNK // 16):
                sl = pl.ds(16 * k, 16)
                e = (plsc.load_gather(as_v, [srcb[q2, sl]])
                     + plsc.load_gather(ad_v, [dstb[q2, sl]]))
                wb[q2, sl] = jnp.exp(jnp.maximum(e, 0.2 * e))

        pltpu.async_copy(h_hbm.at[srcb.at[0]], rows0, gs0)

        @pl.loop(0, NB, step=2)
        def _(q2):
            pltpu.async_copy(h_hbm.at[srcb.at[q2 + 1]], rows1, gs1)
            pltpu.make_async_copy(h_hbm.at[srcb.at[q2]], rows0, gs0).wait()
            _scale(rows0, wb.at[q2])
            _scatter(rows0, q2)

            @pl.when(q2 + 2 < NB)
            def _():
                pltpu.async_copy(h_hbm.at[srcb.at[q2 + 2]], rows0, gs0)

            pltpu.make_async_copy(h_hbm.at[srcb.at[q2 + 1]], rows1, gs1).wait()
            _scale(rows1, wb.at[q2 + 1])
            _scatter(rows1, q2 + 1)

    plsc.subcore_barrier()

    pltpu.sync_copy(accS.at[pl.ds(row0, ROWS_PER_TILE)],
                    acc_out.at[cid, pl.ds(row0, ROWS_PER_TILE)])
    pltpu.sync_copy(denS.at[pl.ds(row0, ROWS_PER_TILE)],
                    den_out.at[cid, pl.ds(row0, ROWS_PER_TILE)])


def kernel(x, edge_index, W1, as1, ad1, b1, W2, as2, ad2, b2, W3, as3, ad3, b3):
    ei = edge_index.astype(jnp.int32)
    pad = jnp.full((E_PAD - E,), N, jnp.int32)
    src = jnp.concatenate([ei[0], pad]).reshape(NCHUNKS, CHUNK)
    dst = jnp.concatenate([ei[1], pad]).reshape(NCHUNKS, CHUNK)

    xp = jnp.pad(x, ((0, N_PAD - N), (0, 0)))

    def edge_phase(h, a_s, a_d):
        acc, den = _sc_edge_agg(h, a_s.reshape(N_PAD), a_d.reshape(N_PAD),
                                src, dst)
        return acc, den.reshape(NC, N_PAD, 1)

    def r2(v):
        return v.reshape(1, D)

    h1, s1, d1 = _dense_first(xp, W1, r2(as1), r2(ad1))
    acc1, den1 = edge_phase(h1, s1, d1)
    h2, s2, d2 = _dense_mid(acc1, den1, h1, s1, d1, r2(b1), W2, r2(as2), r2(ad2))
    acc2, den2 = edge_phase(h2, s2, d2)
    h3, s3, d3 = _dense_mid(acc2, den2, h2, s2, d2, r2(b2), W3, r2(as3), r2(ad3))
    acc3, den3 = edge_phase(h3, s3, d3)
    (out,) = _dense_last(acc3, den3, h3, s3, d3, r2(b3))
    return out[:N]


# speedup vs baseline: 24.1646x; 1.0006x over previous
import functools

import jax
import jax.numpy as jnp
from jax import lax
from jax.experimental import pallas as pl
from jax.experimental.pallas import tpu as pltpu
from jax.experimental.pallas import tpu_sc as plsc

N = 10000
E = 320000
D = 128

N_PAD = 10240
NC = 2
NS = 16
NW = NC * NS
CHUNK = 64
NB = 16
NBLK0 = 13
NBLK1 = 7
ROWS_PER_TILE = N_PAD // NS
E_PAD = NS * (NBLK0 + NBLK1) * NB * CHUNK
NCHUNKS = E_PAD // CHUNK
RB = 1024
GRID = N_PAD // RB


def _dense_first_body(x_ref, w_ref, ats_ref, atd_ref, h_ref, as_ref, ad_ref):
    h = jnp.dot(x_ref[...], w_ref[...], preferred_element_type=jnp.float32)
    h_ref[...] = h
    as_ref[...] = jnp.sum(h * ats_ref[...], axis=1, keepdims=True)
    ad_ref[...] = jnp.sum(h * atd_ref[...], axis=1, keepdims=True)


def _dense_first(x, W, ats, atd):
    return pl.pallas_call(
        _dense_first_body,
        grid=(GRID,),
        in_specs=[
            pl.BlockSpec((RB, D), lambda i: (i, 0)),
            pl.BlockSpec((D, D), lambda i: (0, 0)),
            pl.BlockSpec((1, D), lambda i: (0, 0)),
            pl.BlockSpec((1, D), lambda i: (0, 0)),
        ],
        out_specs=[
            pl.BlockSpec((RB, D), lambda i: (i, 0)),
            pl.BlockSpec((RB, 1), lambda i: (i, 0)),
            pl.BlockSpec((RB, 1), lambda i: (i, 0)),
        ],
        out_shape=[
            jax.ShapeDtypeStruct((N_PAD, D), jnp.float32),
            jax.ShapeDtypeStruct((N_PAD, 1), jnp.float32),
            jax.ShapeDtypeStruct((N_PAD, 1), jnp.float32),
        ],
    )(x, W, ats, atd)


def _finish_node(acc_ref, den_ref, hp_ref, asp_ref, adp_ref, b_ref):
    e = asp_ref[...] + adp_ref[...]
    ws = jnp.exp(jnp.maximum(e, 0.2 * e))
    num = acc_ref[0] + acc_ref[1] + ws * hp_ref[...]
    den = den_ref[0] + den_ref[1] + ws
    return num / den + b_ref[...]


def _dense_mid_body(acc_ref, den_ref, hp_ref, asp_ref, adp_ref, b_ref,
                    w_ref, ats_ref, atd_ref, h_ref, as_ref, ad_ref):
    x = jnp.maximum(_finish_node(acc_ref, den_ref, hp_ref, asp_ref, adp_ref, b_ref), 0.0)
    h = jnp.dot(x, w_ref[...], preferred_element_type=jnp.float32)
    h_ref[...] = h
    as_ref[...] = jnp.sum(h * ats_ref[...], axis=1, keepdims=True)
    ad_ref[...] = jnp.sum(h * atd_ref[...], axis=1, keepdims=True)


def _dense_mid(acc, den, hp, asp, adp, b, W, ats, atd):
    return pl.pallas_call(
        _dense_mid_body,
        grid=(GRID,),
        in_specs=[
            pl.BlockSpec((NC, RB, D), lambda i: (0, i, 0)),
            pl.BlockSpec((NC, RB, 1), lambda i: (0, i, 0)),
            pl.BlockSpec((RB, D), lambda i: (i, 0)),
            pl.BlockSpec((RB, 1), lambda i: (i, 0)),
            pl.BlockSpec((RB, 1), lambda i: (i, 0)),
            pl.BlockSpec((1, D), lambda i: (0, 0)),
            pl.BlockSpec((D, D), lambda i: (0, 0)),
            pl.BlockSpec((1, D), lambda i: (0, 0)),
            pl.BlockSpec((1, D), lambda i: (0, 0)),
        ],
        out_specs=[
            pl.BlockSpec((RB, D), lambda i: (i, 0)),
            pl.BlockSpec((RB, 1), lambda i: (i, 0)),
            pl.BlockSpec((RB, 1), lambda i: (i, 0)),
        ],
        out_shape=[
            jax.ShapeDtypeStruct((N_PAD, D), jnp.float32),
            jax.ShapeDtypeStruct((N_PAD, 1), jnp.float32),
            jax.ShapeDtypeStruct((N_PAD, 1), jnp.float32),
        ],
    )(acc, den, hp, asp, adp, b, W, ats, atd)


def _dense_last_body(acc_ref, den_ref, hp_ref, asp_ref, adp_ref, b_ref, out_ref):
    out_ref[...] = _finish_node(acc_ref, den_ref, hp_ref, asp_ref, adp_ref, b_ref)


def _dense_last(acc, den, hp, asp, adp, b):
    return pl.pallas_call(
        _dense_last_body,
        grid=(GRID,),
        in_specs=[
            pl.BlockSpec((NC, RB, D), lambda i: (0, i, 0)),
            pl.BlockSpec((NC, RB, 1), lambda i: (0, i, 0)),
            pl.BlockSpec((RB, D), lambda i: (i, 0)),
            pl.BlockSpec((RB, 1), lambda i: (i, 0)),
            pl.BlockSpec((RB, 1), lambda i: (i, 0)),
            pl.BlockSpec((1, D), lambda i: (0, 0)),
        ],
        out_specs=[pl.BlockSpec((RB, D), lambda i: (i, 0))],
        out_shape=[jax.ShapeDtypeStruct((N_PAD, D), jnp.float32)],
    )(acc, den, hp, asp, adp, b)


_SC_MESH = plsc.VectorSubcoreMesh(core_axis_name="c", subcore_axis_name="s")


@functools.partial(
    pl.kernel,
    out_type=[
        jax.ShapeDtypeStruct((NC, N_PAD, D), jnp.float32),
        jax.ShapeDtypeStruct((NC, N_PAD), jnp.float32),
    ],
    mesh=_SC_MESH,
    compiler_params=pltpu.CompilerParams(needs_layout_passes=False),
    scratch_types=[
        pltpu.VMEM((N_PAD,), jnp.float32),
        pltpu.VMEM((N_PAD,), jnp.float32),
        pltpu.VMEM((NB, CHUNK), jnp.int32),
        pltpu.VMEM((NB, CHUNK), jnp.int32),
        pltpu.VMEM((NB, CHUNK), jnp.float32),
        pltpu.VMEM((CHUNK, D), jnp.float32),
        pltpu.VMEM((CHUNK, D), jnp.float32),
        pltpu.VMEM_SHARED((N_PAD, D), jnp.float32),
        pltpu.VMEM_SHARED((N_PAD,), jnp.float32),
        pltpu.SemaphoreType.DMA,
        pltpu.SemaphoreType.DMA,
    ],
)
def _sc_edge_agg(h_hbm, as_hbm, ad_hbm, src_hbm, dst_hbm,
                 acc_out, den_out,
                 as_v, ad_v, srcb, dstb, wb, rows0, rows1, accS, denS,
                 gs0, gs1):
    cid = lax.axis_index("c")
    sid = lax.axis_index("s")
    wid = cid * NS + sid
    zero16 = jnp.zeros((16,), jnp.float32)

    with jax.named_scope("sc_stage_alphas"):
        pltpu.sync_copy(as_hbm, as_v)
        pltpu.sync_copy(ad_hbm, ad_v)

    with jax.named_scope("sc_zero"):
        @pl.loop(0, CHUNK)
        def _(r):
            for c in range(D // 16):
                rows0[r, pl.ds(16 * c, 16)] = zero16

        row0 = sid * ROWS_PER_TILE
        for k in range(ROWS_PER_TILE // CHUNK):
            pltpu.sync_copy(rows0, accS.at[pl.ds(row0 + k * CHUNK, CHUNK)])
        for k in range(ROWS_PER_TILE // D):
            pltpu.sync_copy(rows0.at[0], denS.at[pl.ds(row0 + k * D, D)])

    plsc.subcore_barrier()

    def _scale(rows, wrow):
        @pl.loop(0, CHUNK, unroll=2)
        def _(j):
            wj = plsc.load_gather(wrow, [jnp.full((16,), j, jnp.int32)])
            for c in range(D // 16):
                sl = pl.ds(16 * c, 16)
                rows[j, sl] = rows[j, sl] * wj

    def _scatter(rows, q2):
        pltpu.sync_copy(rows, accS.at[dstb.at[q2]], add=True)
        pltpu.sync_copy(wb.at[q2], denS.at[dstb.at[q2]], add=True)

    nblk = lax.select(cid == 0, NBLK0, NBLK1)
    blk0 = lax.select(cid == 0, sid * NBLK0, NS * NBLK0 + sid * NBLK1)

    @pl.loop(0, nblk)
    def _(b):
        blk = blk0 + b
        with jax.named_scope("sc_idx"):
            pltpu.sync_copy(src_hbm.at[pl.ds(blk * NB, NB)], srcb)
            pltpu.sync_copy(dst_hbm.at[pl.ds(blk * NB, NB)], dstb)

        with jax.named_scope("sc_wcompute"):
            @pl.loop(0, NB)
            def _(q2):
                for k in range(CHUNK // 16):
                    sl = pl.ds(16 * k, 16)
                    e = (plsc.load_gather(as_v, [srcb[q2, sl]])
                         + plsc.load_gather(ad_v, [dstb[q2, sl]]))
                    wb[q2, sl] = jnp.exp(jnp.maximum(e, 0.2 * e))

        with jax.named_scope("sc_pipeline"):
            pltpu.async_copy(h_hbm.at[srcb.at[0]], rows0, gs0)

            @pl.loop(0, NB, step=2)
            def _(q2):
                pltpu.async_copy(h_hbm.at[srcb.at[q2 + 1]], rows1, gs1)
                pltpu.make_async_copy(h_hbm.at[srcb.at[q2]], rows0, gs0).wait()
                _scale(rows0, wb.at[q2])
                _scatter(rows0, q2)

                @pl.when(q2 + 2 < NB)
                def _():
                    pltpu.async_copy(h_hbm.at[srcb.at[q2 + 2]], rows0, gs0)

                pltpu.make_async_copy(h_hbm.at[srcb.at[q2 + 1]], rows1, gs1).wait()
                _scale(rows1, wb.at[q2 + 1])
                _scatter(rows1, q2 + 1)

    plsc.subcore_barrier()

    with jax.named_scope("sc_writeback"):
        pltpu.sync_copy(accS.at[pl.ds(row0, ROWS_PER_TILE)],
                        acc_out.at[cid, pl.ds(row0, ROWS_PER_TILE)])
        pltpu.sync_copy(denS.at[pl.ds(row0, ROWS_PER_TILE)],
                        den_out.at[cid, pl.ds(row0, ROWS_PER_TILE)])


def kernel(x, edge_index, W1, as1, ad1, b1, W2, as2, ad2, b2, W3, as3, ad3, b3):
    ei = edge_index.astype(jnp.int32)
    pad = jnp.full((E_PAD - E,), N, jnp.int32)
    src = jnp.concatenate([ei[0], pad]).reshape(NCHUNKS, CHUNK)
    dst = jnp.concatenate([ei[1], pad]).reshape(NCHUNKS, CHUNK)

    xp = jnp.pad(x, ((0, N_PAD - N), (0, 0)))

    def edge_phase(h, a_s, a_d):
        acc, den = _sc_edge_agg(h, a_s.reshape(N_PAD), a_d.reshape(N_PAD),
                                src, dst)
        return acc, den.reshape(NC, N_PAD, 1)

    def r2(v):
        return v.reshape(1, D)

    h1, s1, d1 = _dense_first(xp, W1, r2(as1), r2(ad1))
    acc1, den1 = edge_phase(h1, s1, d1)
    h2, s2, d2 = _dense_mid(acc1, den1, h1, s1, d1, r2(b1), W2, r2(as2), r2(ad2))
    acc2, den2 = edge_phase(h2, s2, d2)
    h3, s3, d3 = _dense_mid(acc2, den2, h2, s2, d2, r2(b2), W3, r2(as3), r2(ad3))
    acc3, den3 = edge_phase(h3, s3, d3)
    (out,) = _dense_last(acc3, den3, h3, s3, d3, r2(b3))
    return out[:N]


# speedup vs baseline: 40.8084x; 1.6888x over previous
import functools

import jax
import jax.numpy as jnp
from jax import lax
from jax.experimental import pallas as pl
from jax.experimental.pallas import tpu as pltpu
from jax.experimental.pallas import tpu_sc as plsc

N = 10000
E = 320000
D = 128

N_PAD = 10240
NC = 2
NS = 16
NW = NC * NS
CHUNK = 64
NB = 16
NBLK0 = 10
NBLK1 = 10
ROWS_PER_TILE = N_PAD // NS
E_PAD = NS * (NBLK0 + NBLK1) * NB * CHUNK
NCHUNKS = E_PAD // CHUNK
RB = 1024
GRID = N_PAD // RB


def _dense_first_body(x_ref, w_ref, ats_ref, atd_ref, h_ref, as_ref, ad_ref):
    h = jnp.dot(x_ref[...], w_ref[...], preferred_element_type=jnp.float32)
    h_ref[...] = h
    as_ref[...] = jnp.sum(h * ats_ref[...], axis=1, keepdims=True)
    ad_ref[...] = jnp.sum(h * atd_ref[...], axis=1, keepdims=True)


def _dense_first(x, W, ats, atd):
    return pl.pallas_call(
        _dense_first_body,
        grid=(GRID,),
        in_specs=[
            pl.BlockSpec((RB, D), lambda i: (i, 0)),
            pl.BlockSpec((D, D), lambda i: (0, 0)),
            pl.BlockSpec((1, D), lambda i: (0, 0)),
            pl.BlockSpec((1, D), lambda i: (0, 0)),
        ],
        out_specs=[
            pl.BlockSpec((RB, D), lambda i: (i, 0)),
            pl.BlockSpec((RB, 1), lambda i: (i, 0)),
            pl.BlockSpec((RB, 1), lambda i: (i, 0)),
        ],
        out_shape=[
            jax.ShapeDtypeStruct((N_PAD, D), jnp.float32),
            jax.ShapeDtypeStruct((N_PAD, 1), jnp.float32),
            jax.ShapeDtypeStruct((N_PAD, 1), jnp.float32),
        ],
    )(x, W, ats, atd)


def _finish_node(acc_ref, den_ref, hp_ref, asp_ref, adp_ref, b_ref):
    e = asp_ref[...] + adp_ref[...]
    ws = jnp.exp(jnp.maximum(e, 0.2 * e))
    num = acc_ref[0] + acc_ref[1] + ws * hp_ref[...]
    den = den_ref[0] + den_ref[1] + ws
    return num / den + b_ref[...]


def _dense_mid_body(acc_ref, den_ref, hp_ref, asp_ref, adp_ref, b_ref,
                    w_ref, ats_ref, atd_ref, h_ref, as_ref, ad_ref):
    x = jnp.maximum(_finish_node(acc_ref, den_ref, hp_ref, asp_ref, adp_ref, b_ref), 0.0)
    h = jnp.dot(x, w_ref[...], preferred_element_type=jnp.float32)
    h_ref[...] = h
    as_ref[...] = jnp.sum(h * ats_ref[...], axis=1, keepdims=True)
    ad_ref[...] = jnp.sum(h * atd_ref[...], axis=1, keepdims=True)


def _dense_mid(acc, den, hp, asp, adp, b, W, ats, atd):
    return pl.pallas_call(
        _dense_mid_body,
        grid=(GRID,),
        in_specs=[
            pl.BlockSpec((NC, RB, D), lambda i: (0, i, 0)),
            pl.BlockSpec((NC, RB, 1), lambda i: (0, i, 0)),
            pl.BlockSpec((RB, D), lambda i: (i, 0)),
            pl.BlockSpec((RB, 1), lambda i: (i, 0)),
            pl.BlockSpec((RB, 1), lambda i: (i, 0)),
            pl.BlockSpec((1, D), lambda i: (0, 0)),
            pl.BlockSpec((D, D), lambda i: (0, 0)),
            pl.BlockSpec((1, D), lambda i: (0, 0)),
            pl.BlockSpec((1, D), lambda i: (0, 0)),
        ],
        out_specs=[
            pl.BlockSpec((RB, D), lambda i: (i, 0)),
            pl.BlockSpec((RB, 1), lambda i: (i, 0)),
            pl.BlockSpec((RB, 1), lambda i: (i, 0)),
        ],
        out_shape=[
            jax.ShapeDtypeStruct((N_PAD, D), jnp.float32),
            jax.ShapeDtypeStruct((N_PAD, 1), jnp.float32),
            jax.ShapeDtypeStruct((N_PAD, 1), jnp.float32),
        ],
    )(acc, den, hp, asp, adp, b, W, ats, atd)


def _dense_last_body(acc_ref, den_ref, hp_ref, asp_ref, adp_ref, b_ref, out_ref):
    out_ref[...] = _finish_node(acc_ref, den_ref, hp_ref, asp_ref, adp_ref, b_ref)


def _dense_last(acc, den, hp, asp, adp, b):
    return pl.pallas_call(
        _dense_last_body,
        grid=(GRID,),
        in_specs=[
            pl.BlockSpec((NC, RB, D), lambda i: (0, i, 0)),
            pl.BlockSpec((NC, RB, 1), lambda i: (0, i, 0)),
            pl.BlockSpec((RB, D), lambda i: (i, 0)),
            pl.BlockSpec((RB, 1), lambda i: (i, 0)),
            pl.BlockSpec((RB, 1), lambda i: (i, 0)),
            pl.BlockSpec((1, D), lambda i: (0, 0)),
        ],
        out_specs=[pl.BlockSpec((RB, D), lambda i: (i, 0))],
        out_shape=[jax.ShapeDtypeStruct((N_PAD, D), jnp.float32)],
    )(acc, den, hp, asp, adp, b)


_SC_MESH = plsc.VectorSubcoreMesh(core_axis_name="c", subcore_axis_name="s")


@functools.partial(
    pl.kernel,
    out_type=[
        jax.ShapeDtypeStruct((NC, N_PAD, D), jnp.float32),
        jax.ShapeDtypeStruct((NC, N_PAD), jnp.float32),
    ],
    mesh=_SC_MESH,
    compiler_params=pltpu.CompilerParams(needs_layout_passes=False),
    scratch_types=[
        pltpu.VMEM((N_PAD,), jnp.float32),
        pltpu.VMEM((N_PAD,), jnp.float32),
        pltpu.VMEM((NB, CHUNK), jnp.int32),
        pltpu.VMEM((NB, CHUNK), jnp.int32),
        pltpu.VMEM((NB, CHUNK), jnp.float32),
        pltpu.VMEM((CHUNK, D), jnp.float32),
        pltpu.VMEM((CHUNK, D), jnp.float32),
        pltpu.VMEM_SHARED((N_PAD, D), jnp.float32),
        pltpu.VMEM_SHARED((N_PAD,), jnp.float32),
        pltpu.SemaphoreType.DMA,
        pltpu.SemaphoreType.DMA,
    ],
)
def _sc_edge_agg(h_hbm, as_hbm, ad_hbm, src_hbm, dst_hbm,
                 acc_out, den_out,
                 as_v, ad_v, srcb, dstb, wb, rows0, rows1, accS, denS,
                 gs0, gs1):
    cid = lax.axis_index("c")
    sid = lax.axis_index("s")
    wid = cid * NS + sid
    zero16 = jnp.zeros((16,), jnp.float32)

    with jax.named_scope("sc_stage_alphas"):
        pltpu.sync_copy(as_hbm, as_v)
        pltpu.sync_copy(ad_hbm, ad_v)

    with jax.named_scope("sc_zero"):
        @pl.loop(0, CHUNK)
        def _(r):
            for c in range(D // 16):
                rows0[r, pl.ds(16 * c, 16)] = zero16

        row0 = sid * ROWS_PER_TILE
        for k in range(ROWS_PER_TILE // CHUNK):
            pltpu.sync_copy(rows0, accS.at[pl.ds(row0 + k * CHUNK, CHUNK)])
        for k in range(ROWS_PER_TILE // D):
            pltpu.sync_copy(rows0.at[0], denS.at[pl.ds(row0 + k * D, D)])

    plsc.subcore_barrier()

    def _scale(rows, wrow):
        @pl.loop(0, CHUNK, unroll=2)
        def _(j):
            wj = plsc.load_gather(wrow, [jnp.full((16,), j, jnp.int32)])
            for c in range(D // 16):
                sl = pl.ds(16 * c, 16)
                rows[j, sl] = rows[j, sl] * wj

    def _scatter(rows, q2):
        pltpu.sync_copy(rows, accS.at[dstb.at[q2]], add=True)
        pltpu.sync_copy(wb.at[q2], denS.at[dstb.at[q2]], add=True)

    nblk = lax.select(cid == 0, NBLK0, NBLK1)
    blk0 = lax.select(cid == 0, sid * NBLK0, NS * NBLK0 + sid * NBLK1)

    @pl.loop(0, nblk)
    def _(b):
        blk = blk0 + b
        with jax.named_scope("sc_idx"):
            pltpu.sync_copy(src_hbm.at[pl.ds(blk * NB, NB)], srcb)
            pltpu.sync_copy(dst_hbm.at[pl.ds(blk * NB, NB)], dstb)

        with jax.named_scope("sc_wcompute"):
            @pl.loop(0, NB)
            def _(q2):
                for k in range(CHUNK // 16):
                    sl = pl.ds(16 * k, 16)
                    e = (plsc.load_gather(as_v, [srcb[q2, sl]])
                         + plsc.load_gather(ad_v, [dstb[q2, sl]]))
                    wb[q2, sl] = jnp.exp(jnp.maximum(e, 0.2 * e))

        with jax.named_scope("sc_pipeline"):
            pltpu.async_copy(h_hbm.at[srcb.at[0]], rows0, gs0)

            @pl.loop(0, NB, step=2)
            def _(q2):
                pltpu.async_copy(h_hbm.at[srcb.at[q2 + 1]], rows1, gs1)
                pltpu.make_async_copy(h_hbm.at[srcb.at[q2]], rows0, gs0).wait()
                _scale(rows0, wb.at[q2])
                _scatter(rows0, q2)

                @pl.when(q2 + 2 < NB)
                def _():
                    pltpu.async_copy(h_hbm.at[srcb.at[q2 + 2]], rows0, gs0)

                pltpu.make_async_copy(h_hbm.at[srcb.at[q2 + 1]], rows1, gs1).wait()
                _scale(rows1, wb.at[q2 + 1])
                _scatter(rows1, q2 + 1)

    plsc.subcore_barrier()

    with jax.named_scope("sc_writeback"):
        pltpu.sync_copy(accS.at[pl.ds(row0, ROWS_PER_TILE)],
                        acc_out.at[cid, pl.ds(row0, ROWS_PER_TILE)])
        pltpu.sync_copy(denS.at[pl.ds(row0, ROWS_PER_TILE)],
                        den_out.at[cid, pl.ds(row0, ROWS_PER_TILE)])


def kernel(x, edge_index, W1, as1, ad1, b1, W2, as2, ad2, b2, W3, as3, ad3, b3):
    ei = edge_index.astype(jnp.int32)
    pad = N + jnp.arange(E_PAD - E, dtype=jnp.int32) % (N_PAD - N)
    src = jnp.concatenate([ei[0], pad]).reshape(NCHUNKS, CHUNK)
    dst = jnp.concatenate([ei[1], pad]).reshape(NCHUNKS, CHUNK)

    xp = jnp.pad(x, ((0, N_PAD - N), (0, 0)))

    def edge_phase(h, a_s, a_d):
        acc, den = _sc_edge_agg(h, a_s.reshape(N_PAD), a_d.reshape(N_PAD),
                                src, dst)
        return acc, den.reshape(NC, N_PAD, 1)

    def r2(v):
        return v.reshape(1, D)

    h1, s1, d1 = _dense_first(xp, W1, r2(as1), r2(ad1))
    acc1, den1 = edge_phase(h1, s1, d1)
    h2, s2, d2 = _dense_mid(acc1, den1, h1, s1, d1, r2(b1), W2, r2(as2), r2(ad2))
    acc2, den2 = edge_phase(h2, s2, d2)
    h3, s3, d3 = _dense_mid(acc2, den2, h2, s2, d2, r2(b2), W3, r2(as3), r2(ad3))
    acc3, den3 = edge_phase(h3, s3, d3)
    (out,) = _dense_last(acc3, den3, h3, s3, d3, r2(b3))
    return out[:N]
